# baseline (speedup 1.0000x reference)
import functools

import jax
import jax.numpy as jnp
from jax import lax
from jax.experimental import pallas as pl
from jax.experimental.pallas import tpu as pltpu
from jax.experimental.pallas import tpu_sc as plsc

B, S, H, D = 1, 2048, 12, 64
DMODEL = H * D
DFF = 2048
E = 8
T = B * S
TOPK = 2

BT = 256
NTILES = (T * TOPK + E * (BT - 1) + BT - 1) // BT
LPAD = NTILES * BT
BF = 1024
NJ = DFF // BF

NC, NS = 2, 16
NW = NC * NS


def _router_kernel(x_ref, gw_ref, w_ref, idx_ref):
    x = x_ref[...]
    logits = jnp.dot(x, gw_ref[...], preferred_element_type=jnp.float32)
    am1 = jnp.argmax(logits, axis=1)[:, None]
    eids = jax.lax.broadcasted_iota(jnp.int32, logits.shape, 1)
    m1 = jnp.max(logits, axis=1, keepdims=True)
    masked = jnp.where(eids == am1, -jnp.inf, logits)
    am2 = jnp.argmax(masked, axis=1)[:, None]
    m2 = jnp.max(masked, axis=1, keepdims=True)
    w1v = 1.0 / (1.0 + jnp.exp(m2 - m1))
    w_ref[...] = jnp.concatenate([w1v, 1.0 - w1v], axis=1)
    idx_ref[...] = jnp.concatenate([am1, am2], axis=1).astype(jnp.int32)


def _router(x, gate_W):
    return pl.pallas_call(
        _router_kernel,
        out_shape=(jax.ShapeDtypeStruct((T, TOPK), jnp.float32),
                   jax.ShapeDtypeStruct((T, TOPK), jnp.int32)),
    )(x, gate_W)


def _dispatch(x, tok_padded):
    rows_w = LPAD // NW
    chunk = rows_w // 2

    mesh = plsc.VectorSubcoreMesh(core_axis_name="c", subcore_axis_name="s")

    @functools.partial(
        pl.kernel, mesh=mesh,
        out_type=jax.ShapeDtypeStruct((LPAD, DMODEL), jnp.float32),
        scratch_types=[
            pltpu.VMEM((chunk,), jnp.int32),
            pltpu.VMEM((chunk, DMODEL), jnp.float32),
            pltpu.SemaphoreType.DMA,
        ],
    )
    def k(x_hbm, tok_hbm, xg_hbm, idx_v, rows_v, sem):
        wid = lax.axis_index("s") * NC + lax.axis_index("c")
        base = wid * rows_w
        for c in range(2):
            off = base + c * chunk
            pltpu.sync_copy(tok_hbm.at[pl.ds(off, chunk)], idx_v)
            pltpu.async_copy(x_hbm.at[idx_v], rows_v, sem).wait()
            pltpu.sync_copy(rows_v, xg_hbm.at[pl.ds(off, chunk)])

    return k(x, tok_padded)


def _ffn_kernel(te_ref, xg_ref, w1_ref, w3_ref, w2_ref, out_ref):
    j = pl.program_id(0)
    tl = pl.program_id(1)
    nvalid = te_ref[NTILES]

    @pl.when(tl < nvalid)
    def _():
        xb = xg_ref[pl.ds(tl * BT, BT), :].astype(jnp.bfloat16)
        g = jnp.dot(xb, w1_ref[0], preferred_element_type=jnp.float32)
        u = jnp.dot(xb, w3_ref[0], preferred_element_type=jnp.float32)
        g = g * jax.nn.sigmoid(g)
        h = (g * u).astype(jnp.bfloat16)
        part = jnp.dot(h, w2_ref[0], preferred_element_type=jnp.float32)
        sl = pl.ds(tl * BT, BT)

        @pl.when(j == 0)
        def _():
            out_ref[sl, :] = part

        @pl.when(j != 0)
        def _():
            out_ref[sl, :] += part


def _ffn(scalars, xg, W1, W3, W2):
    grid_spec = pltpu.PrefetchScalarGridSpec(
        num_scalar_prefetch=1,
        grid=(NJ, NTILES),
        in_specs=[
            pl.BlockSpec((LPAD, DMODEL), lambda j, tl, te: (0, 0)),
            pl.BlockSpec((1, DMODEL, BF), lambda j, tl, te: (te[tl], 0, j)),
            pl.BlockSpec((1, DMODEL, BF), lambda j, tl, te: (te[tl], 0, j)),
            pl.BlockSpec((1, BF, DMODEL), lambda j, tl, te: (te[tl], j, 0)),
        ],
        out_specs=pl.BlockSpec((LPAD, DMODEL), lambda j, tl, te: (0, 0)),
    )
    return pl.pallas_call(
        _ffn_kernel,
        grid_spec=grid_spec,
        out_shape=jax.ShapeDtypeStruct((LPAD, DMODEL), jnp.float32),
        compiler_params=pltpu.CompilerParams(
            dimension_semantics=("arbitrary", "arbitrary"),
        ),
    )(scalars, xg, W1, W3, W2)


def _combine_gather(yg, pos0, pos1):
    rows_w = T // NW

    mesh = plsc.VectorSubcoreMesh(core_axis_name="c", subcore_axis_name="s")

    @functools.partial(
        pl.kernel, mesh=mesh,
        out_type=(jax.ShapeDtypeStruct((T, DMODEL), jnp.float32),
                  jax.ShapeDtypeStruct((T, DMODEL), jnp.float32)),
        scratch_types=[
            pltpu.VMEM((rows_w,), jnp.int32),
            pltpu.VMEM((rows_w, DMODEL), jnp.float32),
            pltpu.SemaphoreType.DMA,
        ],
    )
    def k(yg_hbm, p0_hbm, p1_hbm, y0_hbm, y1_hbm, idx_v, rows_v, sem):
        wid = lax.axis_index("s") * NC + lax.axis_index("c")
        base = wid * rows_w
        for p_hbm, y_hbm in ((p0_hbm, y0_hbm), (p1_hbm, y1_hbm)):
            pltpu.sync_copy(p_hbm.at[pl.ds(base, rows_w)], idx_v)
            pltpu.async_copy(yg_hbm.at[idx_v], rows_v, sem).wait()
            pltpu.sync_copy(rows_v, y_hbm.at[pl.ds(base, rows_w)])

    return k(yg, pos0, pos1)


def _add_kernel(w_ref, y0_ref, y1_ref, o_ref):
    w = w_ref[...]
    o_ref[...] = w[:, 0:1] * y0_ref[...] + w[:, 1:2] * y1_ref[...]


def _combine_add(gate_w, y0, y1):
    nb = 4
    rb = T // nb
    return pl.pallas_call(
        _add_kernel,
        grid=(nb,),
        in_specs=[
            pl.BlockSpec((rb, TOPK), lambda i: (i, 0)),
            pl.BlockSpec((rb, DMODEL), lambda i: (i, 0)),
            pl.BlockSpec((rb, DMODEL), lambda i: (i, 0)),
        ],
        out_specs=pl.BlockSpec((rb, DMODEL), lambda i: (i, 0)),
        out_shape=jax.ShapeDtypeStruct((T, DMODEL), jnp.float32),
    )(gate_w, y0, y1)


@jax.jit
def _moe(x, gate_W, W1, W2, W3):
    gate_w, gate_idx = _router(x, gate_W)

    eid = gate_idx.reshape(-1)
    oh = (eid[:, None] == jnp.arange(E, dtype=jnp.int32)[None, :])
    oh = oh.astype(jnp.int32)
    counts = oh.sum(axis=0)
    rank = jnp.cumsum(oh, axis=0) - oh
    rank_i = (rank * oh).sum(axis=1)
    pc = ((counts + BT - 1) // BT) * BT
    pend = jnp.cumsum(pc)
    pstart = pend - pc
    pos = (pstart[eid] + rank_i).astype(jnp.int32)
    pair_tok = jnp.arange(T * TOPK, dtype=jnp.int32) // TOPK
    pad_fill = jnp.arange(LPAD, dtype=jnp.int32) % T
    tok_padded = pad_fill.at[pos].set(pair_tok)
    nvalid = (pend[-1] // BT).astype(jnp.int32)
    te = (jnp.arange(NTILES, dtype=jnp.int32)[:, None] * BT
          >= pend[None, :]).sum(axis=1)
    te = jnp.minimum(te, E - 1).astype(jnp.int32)
    scalars = jnp.concatenate([te, nvalid[None]])
    posk = pos.reshape(T, TOPK)
    pos0 = posk[:, 0]
    pos1 = posk[:, 1]

    xg = _dispatch(x, tok_padded)
    yg = _ffn(scalars, xg,
              W1.astype(jnp.bfloat16),
              W3.astype(jnp.bfloat16),
              W2.astype(jnp.bfloat16))
    y0, y1 = _combine_gather(yg, pos0, pos1)
    return _combine_add(gate_w, y0, y1)


def kernel(stm, gate_W, W1, W2, W3):
    b, s, h, dh = stm.shape
    x = stm.reshape(b * s, h * dh)
    out = _moe(x, gate_W, W1, W2, W3)
    return out.reshape(b, s, h, dh)


# speedup vs baseline: 1.1773x; 1.1773x over previous
import functools

import jax
import jax.numpy as jnp
from jax import lax
from jax.experimental import pallas as pl
from jax.experimental.pallas import tpu as pltpu
from jax.experimental.pallas import tpu_sc as plsc

B, S, H, D = 1, 2048, 12, 64
DMODEL = H * D
DFF = 2048
E = 8
T = B * S
TOPK = 2

BT = 256
NTILES = (T * TOPK + E * (BT - 1) + BT - 1) // BT
LPAD = NTILES * BT
BF = 1024
NJ = DFF // BF

NC, NS = 2, 16
NW = NC * NS


def _router_kernel(x_ref, gw_ref, w_ref, idx_ref):
    x = x_ref[...]
    logits = jnp.dot(x, gw_ref[...], preferred_element_type=jnp.float32)
    am1 = jnp.argmax(logits, axis=1)[:, None]
    eids = jax.lax.broadcasted_iota(jnp.int32, logits.shape, 1)
    m1 = jnp.max(logits, axis=1, keepdims=True)
    masked = jnp.where(eids == am1, -jnp.inf, logits)
    am2 = jnp.argmax(masked, axis=1)[:, None]
    m2 = jnp.max(masked, axis=1, keepdims=True)
    w1v = 1.0 / (1.0 + jnp.exp(m2 - m1))
    w_ref[...] = jnp.concatenate([w1v, 1.0 - w1v], axis=1)
    idx_ref[...] = jnp.concatenate([am1, am2], axis=1).astype(jnp.int32)


def _router(x, gate_W):
    return pl.pallas_call(
        _router_kernel,
        out_shape=(jax.ShapeDtypeStruct((T, TOPK), jnp.float32),
                   jax.ShapeDtypeStruct((T, TOPK), jnp.int32)),
    )(x, gate_W)


def _dispatch(x, tok_padded):
    rows_w = LPAD // NW
    chunk = rows_w // 2

    mesh = plsc.VectorSubcoreMesh(core_axis_name="c", subcore_axis_name="s")

    @functools.partial(
        pl.kernel, mesh=mesh,
        out_type=jax.ShapeDtypeStruct((LPAD, DMODEL), jnp.float32),
        scratch_types=[
            pltpu.VMEM((chunk,), jnp.int32),
            pltpu.VMEM((chunk, DMODEL), jnp.float32),
            pltpu.SemaphoreType.DMA,
        ],
    )
    def k(x_hbm, tok_hbm, xg_hbm, idx_v, rows_v, sem):
        wid = lax.axis_index("s") * NC + lax.axis_index("c")
        base = wid * rows_w
        for c in range(2):
            off = base + c * chunk
            pltpu.sync_copy(tok_hbm.at[pl.ds(off, chunk)], idx_v)
            pltpu.async_copy(x_hbm.at[idx_v], rows_v, sem).wait()
            pltpu.sync_copy(rows_v, xg_hbm.at[pl.ds(off, chunk)])

    return k(x, tok_padded)


def _ffn_kernel(te_ref, xg_ref, w1_ref, w3_ref, w2_ref, tok_ref, ws_ref,
                out_ref, w1b_ref, w3b_ref, w2b_ref, acc_ref):
    j = pl.program_id(0)
    tl = pl.program_id(1)
    nvalid = te_ref[NTILES]
    prev = te_ref[jnp.maximum(tl - 1, 0)]
    refresh = (tl == 0) | (te_ref[tl] != prev)

    @pl.when((j == 0) & (tl == 0))
    def _():
        out_ref[...] = jnp.zeros_like(out_ref)

    @pl.when(refresh)
    def _():
        w1b_ref[...] = w1_ref[0].astype(jnp.bfloat16)
        w3b_ref[...] = w3_ref[0].astype(jnp.bfloat16)
        w2b_ref[...] = w2_ref[0].astype(jnp.bfloat16)

    @pl.when(tl < nvalid)
    def _():
        xb = xg_ref[...].astype(jnp.bfloat16)
        g = jnp.dot(xb, w1b_ref[...], preferred_element_type=jnp.float32)
        u = jnp.dot(xb, w3b_ref[...], preferred_element_type=jnp.float32)
        g = g * jax.nn.sigmoid(g)
        h = (g * u).astype(jnp.bfloat16)
        part = jnp.dot(h, w2b_ref[...], preferred_element_type=jnp.float32)

        sl = pl.ds(tl * BT, BT)
        if NJ > 1:
            @pl.when(j == 0)
            def _():
                acc_ref[sl, :] = part

            @pl.when((j > 0) & (j < NJ - 1))
            def _():
                acc_ref[sl, :] += part

        @pl.when(j == NJ - 1)
        def _():
            full = part if NJ == 1 else acc_ref[sl, :] + part
            y = (full * ws_ref[0]).astype(jnp.bfloat16)
            tok = tok_ref[0]
            ti = jax.lax.broadcasted_iota(jnp.int32, (T, BT), 0)
            pt = jnp.where(ti == tok, 1.0, 0.0).astype(jnp.bfloat16)
            out_ref[...] += jnp.dot(pt, y, preferred_element_type=jnp.float32)


def _ffn(scalars, xg, W1, W3, W2, tok3, ws3):
    grid_spec = pltpu.PrefetchScalarGridSpec(
        num_scalar_prefetch=1,
        grid=(NJ, NTILES),
        in_specs=[
            pl.BlockSpec((BT, DMODEL), lambda j, tl, te: (tl, 0)),
            pl.BlockSpec((1, DMODEL, BF), lambda j, tl, te: (te[tl], 0, j)),
            pl.BlockSpec((1, DMODEL, BF), lambda j, tl, te: (te[tl], 0, j)),
            pl.BlockSpec((1, BF, DMODEL), lambda j, tl, te: (te[tl], j, 0)),
            pl.BlockSpec((1, 1, BT), lambda j, tl, te: (tl, 0, 0)),
            pl.BlockSpec((1, BT, 1), lambda j, tl, te: (tl, 0, 0)),
        ],
        out_specs=pl.BlockSpec((T, DMODEL), lambda j, tl, te: (0, 0)),
        scratch_shapes=[
            pltpu.VMEM((DMODEL, BF), jnp.bfloat16),
            pltpu.VMEM((DMODEL, BF), jnp.bfloat16),
            pltpu.VMEM((BF, DMODEL), jnp.bfloat16),
            pltpu.VMEM((LPAD, DMODEL), jnp.float32),
        ],
    )
    return pl.pallas_call(
        _ffn_kernel,
        grid_spec=grid_spec,
        out_shape=jax.ShapeDtypeStruct((T, DMODEL), jnp.float32),
        compiler_params=pltpu.CompilerParams(
            dimension_semantics=("arbitrary", "arbitrary"),
        ),
    )(scalars, xg, W1, W3, W2, tok3, ws3)


@jax.jit
def _moe(x, gate_W, W1, W2, W3):
    gate_w, gate_idx = _router(x, gate_W)

    eid = gate_idx.reshape(-1)
    oh = (eid[:, None] == jnp.arange(E, dtype=jnp.int32)[None, :])
    oh = oh.astype(jnp.int32)
    counts = oh.sum(axis=0)
    rank = jnp.cumsum(oh, axis=0) - oh
    rank_i = (rank * oh).sum(axis=1)
    pc = ((counts + BT - 1) // BT) * BT
    pend = jnp.cumsum(pc)
    pstart = pend - pc
    pos = (pstart[eid] + rank_i).astype(jnp.int32)
    pair_tok = jnp.arange(T * TOPK, dtype=jnp.int32) // TOPK
    pad_fill = jnp.arange(LPAD, dtype=jnp.int32) % T
    tok_padded = pad_fill.at[pos].set(pair_tok)
    ws = jnp.zeros((LPAD,), jnp.float32).at[pos].set(gate_w.reshape(-1))
    nvalid = (pend[-1] // BT).astype(jnp.int32)
    te = (jnp.arange(NTILES, dtype=jnp.int32)[:, None] * BT
          >= pend[None, :]).sum(axis=1)
    te = jnp.minimum(te, E - 1).astype(jnp.int32)
    scalars = jnp.concatenate([te, nvalid[None]])
    tok3 = tok_padded.reshape(NTILES, 1, BT)
    ws3 = ws.reshape(NTILES, BT, 1)

    xg = _dispatch(x, tok_padded)
    return _ffn(scalars, xg, W1, W3, W2, tok3, ws3)


def kernel(stm, gate_W, W1, W2, W3):
    b, s, h, dh = stm.shape
    x = stm.reshape(b * s, h * dh)
    out = _moe(x, gate_W, W1, W2, W3)
    return out.reshape(b, s, h, dh)


# speedup vs baseline: 1.2197x; 1.0360x over previous
import functools

import jax
import jax.numpy as jnp
from jax import lax
from jax.experimental import pallas as pl
from jax.experimental.pallas import tpu as pltpu
from jax.experimental.pallas import tpu_sc as plsc

B, S, H, D = 1, 2048, 12, 64
DMODEL = H * D
DFF = 2048
E = 8
T = B * S
TOPK = 2

BT = 256
NTILES = (T * TOPK + E * (BT - 1) + BT - 1) // BT
LPAD = NTILES * BT
BF = 1024
NJ = DFF // BF

NC, NS = 2, 16
NW = NC * NS


def _router_kernel(x_ref, gw_ref, w_ref, idx_ref):
    x = x_ref[...]
    logits = jnp.dot(x, gw_ref[...], preferred_element_type=jnp.float32)
    am1 = jnp.argmax(logits, axis=1)[:, None]
    eids = jax.lax.broadcasted_iota(jnp.int32, logits.shape, 1)
    m1 = jnp.max(logits, axis=1, keepdims=True)
    masked = jnp.where(eids == am1, -jnp.inf, logits)
    am2 = jnp.argmax(masked, axis=1)[:, None]
    m2 = jnp.max(masked, axis=1, keepdims=True)
    w1v = 1.0 / (1.0 + jnp.exp(m2 - m1))
    w_ref[...] = jnp.concatenate([w1v, 1.0 - w1v], axis=1)
    idx_ref[...] = jnp.concatenate([am1, am2], axis=1).astype(jnp.int32)


def _router(x, gate_W):
    return pl.pallas_call(
        _router_kernel,
        out_shape=(jax.ShapeDtypeStruct((T, TOPK), jnp.float32),
                   jax.ShapeDtypeStruct((T, TOPK), jnp.int32)),
    )(x, gate_W)


def _dispatch(x, tok_padded):
    rows_w = LPAD // NW
    chunk = rows_w // 2

    mesh = plsc.VectorSubcoreMesh(core_axis_name="c", subcore_axis_name="s")

    @functools.partial(
        pl.kernel, mesh=mesh,
        out_type=jax.ShapeDtypeStruct((LPAD, DMODEL), jnp.float32),
        scratch_types=[
            pltpu.VMEM((chunk,), jnp.int32),
            pltpu.VMEM((chunk, DMODEL), jnp.float32),
            pltpu.SemaphoreType.DMA,
        ],
    )
    def k(x_hbm, tok_hbm, xg_hbm, idx_v, rows_v, sem):
        wid = lax.axis_index("s") * NC + lax.axis_index("c")
        base = wid * rows_w
        for c in range(2):
            off = base + c * chunk
            pltpu.sync_copy(tok_hbm.at[pl.ds(off, chunk)], idx_v)
            pltpu.async_copy(x_hbm.at[idx_v], rows_v, sem).wait()
            pltpu.sync_copy(rows_v, xg_hbm.at[pl.ds(off, chunk)])

    return k(x, tok_padded)


def _ffn_kernel(te_ref, xg_ref, w1_ref, w3_ref, w2_ref, tokw_ref,
                out_ref, w1b_ref, w3b_ref, w2b_ref, acc_ref):
    j = pl.program_id(0)
    tl = pl.program_id(1)
    nvalid = te_ref[NTILES]
    prev = te_ref[jnp.maximum(tl - 1, 0)]
    refresh = (tl == 0) | (te_ref[tl] != prev)

    @pl.when((j == 0) & (tl == 0))
    def _():
        out_ref[...] = jnp.zeros_like(out_ref)

    @pl.when(refresh)
    def _():
        w1b_ref[...] = w1_ref[0].astype(jnp.bfloat16)
        w3b_ref[...] = w3_ref[0].astype(jnp.bfloat16)
        w2b_ref[...] = w2_ref[0].astype(jnp.bfloat16)

    @pl.when(tl < nvalid)
    def _():
        xb = xg_ref[...].astype(jnp.bfloat16)
        g = jnp.dot(xb, w1b_ref[...], preferred_element_type=jnp.float32)
        u = jnp.dot(xb, w3b_ref[...], preferred_element_type=jnp.float32)
        g = g * jax.nn.sigmoid(g)
        h = (g * u).astype(jnp.bfloat16)
        part = jnp.dot(h, w2b_ref[...], preferred_element_type=jnp.float32)

        sl = pl.ds(tl * BT, BT)
        if NJ > 1:
            @pl.when(j == 0)
            def _():
                acc_ref[sl, :] = part.astype(jnp.bfloat16)

            @pl.when((j > 0) & (j < NJ - 1))
            def _():
                acc_ref[sl, :] += part.astype(jnp.bfloat16)

        @pl.when(j == NJ - 1)
        def _():
            if NJ == 1:
                full = part
            else:
                full = acc_ref[sl, :].astype(jnp.float32) + part
            y = full.astype(jnp.bfloat16)
            v = tokw_ref[0]
            tok = v & 0xFFFF
            wv = (v >> 16).astype(jnp.float32) * (1.0 / 16384.0)
            ti = jax.lax.broadcasted_iota(jnp.int32, (T, BT), 0)
            pt = jnp.where(ti == tok, wv, 0.0).astype(jnp.bfloat16)
            out_ref[...] += jnp.dot(pt, y, preferred_element_type=jnp.float32)


def _ffn(scalars, xg, W1, W3, W2, tokw3):
    grid_spec = pltpu.PrefetchScalarGridSpec(
        num_scalar_prefetch=1,
        grid=(NJ, NTILES),
        in_specs=[
            pl.BlockSpec((BT, DMODEL), lambda j, tl, te: (tl, 0)),
            pl.BlockSpec((1, DMODEL, BF), lambda j, tl, te: (te[tl], 0, j)),
            pl.BlockSpec((1, DMODEL, BF), lambda j, tl, te: (te[tl], 0, j)),
            pl.BlockSpec((1, BF, DMODEL), lambda j, tl, te: (te[tl], j, 0)),
            pl.BlockSpec((1, 1, BT), lambda j, tl, te: (tl, 0, 0)),
        ],
        out_specs=pl.BlockSpec((T, DMODEL), lambda j, tl, te: (0, 0)),
        scratch_shapes=[
            pltpu.VMEM((DMODEL, BF), jnp.bfloat16),
            pltpu.VMEM((DMODEL, BF), jnp.bfloat16),
            pltpu.VMEM((BF, DMODEL), jnp.bfloat16),
            pltpu.VMEM((LPAD, DMODEL), jnp.bfloat16),
        ],
    )
    return pl.pallas_call(
        _ffn_kernel,
        grid_spec=grid_spec,
        out_shape=jax.ShapeDtypeStruct((T, DMODEL), jnp.float32),
        compiler_params=pltpu.CompilerParams(
            dimension_semantics=("arbitrary", "arbitrary"),
        ),
    )(scalars, xg, W1, W3, W2, tokw3)


@jax.jit
def _moe(x, gate_W, W1, W2, W3):
    gate_w, gate_idx = _router(x, gate_W)

    eid = gate_idx.reshape(-1)
    oh = (eid[:, None] == jnp.arange(E, dtype=jnp.int32)[None, :])
    oh = oh.astype(jnp.int32)
    counts = oh.sum(axis=0)
    rank = jnp.cumsum(oh, axis=0) - oh
    rank_i = (rank * oh).sum(axis=1)
    pc = ((counts + BT - 1) // BT) * BT
    pend = jnp.cumsum(pc)
    pstart = pend - pc
    pos = (pstart[eid] + rank_i).astype(jnp.int32)
    pair_tok = jnp.arange(T * TOPK, dtype=jnp.int32) // TOPK
    wq = jnp.round(gate_w.reshape(-1) * 16384.0).astype(jnp.int32)
    packed = (wq << 16) | pair_tok
    pad_fill = jnp.arange(LPAD, dtype=jnp.int32) % T
    tokw = pad_fill.at[pos].set(packed)
    tok_padded = tokw & 0xFFFF
    nvalid = (pend[-1] // BT).astype(jnp.int32)
    te = (jnp.arange(NTILES, dtype=jnp.int32)[:, None] * BT
          >= pend[None, :]).sum(axis=1)
    te = jnp.minimum(te, E - 1).astype(jnp.int32)
    scalars = jnp.concatenate([te, nvalid[None]])
    tokw3 = tokw.reshape(NTILES, 1, BT)

    xg = _dispatch(x, tok_padded)
    return _ffn(scalars, xg, W1, W3, W2, tokw3)


def kernel(stm, gate_W, W1, W2, W3):
    b, s, h, dh = stm.shape
    x = stm.reshape(b * s, h * dh)
    out = _moe(x, gate_W, W1, W2, W3)
    return out.reshape(b, s, h, dh)


# speedup vs baseline: 1.2738x; 1.0443x over previous
import functools

import jax
import jax.numpy as jnp
from jax import lax
from jax.experimental import pallas as pl
from jax.experimental.pallas import tpu as pltpu
from jax.experimental.pallas import tpu_sc as plsc

B, S, H, D = 1, 2048, 12, 64
DMODEL = H * D
DFF = 2048
E = 8
T = B * S
TOPK = 2

BT = 512
NTILES = (T * TOPK + E * (BT - 1) + BT - 1) // BT
LPAD = NTILES * BT
BF = 1024
NJ = DFF // BF

NC, NS = 2, 16
NW = NC * NS


def _router_kernel(x_ref, gw_ref, w_ref, idx_ref):
    x = x_ref[...]
    logits = jnp.dot(x, gw_ref[...], preferred_element_type=jnp.float32)
    am1 = jnp.argmax(logits, axis=1)[:, None]
    eids = jax.lax.broadcasted_iota(jnp.int32, logits.shape, 1)
    m1 = jnp.max(logits, axis=1, keepdims=True)
    masked = jnp.where(eids == am1, -jnp.inf, logits)
    am2 = jnp.argmax(masked, axis=1)[:, None]
    m2 = jnp.max(masked, axis=1, keepdims=True)
    w1v = 1.0 / (1.0 + jnp.exp(m2 - m1))
    w_ref[...] = jnp.concatenate([w1v, 1.0 - w1v], axis=1)
    idx_ref[...] = jnp.concatenate([am1, am2], axis=1).astype(jnp.int32)


def _router(x, gate_W):
    return pl.pallas_call(
        _router_kernel,
        out_shape=(jax.ShapeDtypeStruct((T, TOPK), jnp.float32),
                   jax.ShapeDtypeStruct((T, TOPK), jnp.int32)),
    )(x, gate_W)


def _dispatch(x, tok_padded):
    rows_w = LPAD // NW
    chunk = rows_w // 2

    mesh = plsc.VectorSubcoreMesh(core_axis_name="c", subcore_axis_name="s")

    @functools.partial(
        pl.kernel, mesh=mesh,
        out_type=jax.ShapeDtypeStruct((LPAD, DMODEL), jnp.float32),
        scratch_types=[
            pltpu.VMEM((chunk,), jnp.int32),
            pltpu.VMEM((chunk, DMODEL), jnp.float32),
            pltpu.SemaphoreType.DMA,
        ],
    )
    def k(x_hbm, tok_hbm, xg_hbm, idx_v, rows_v, sem):
        wid = lax.axis_index("s") * NC + lax.axis_index("c")
        base = wid * rows_w
        for c in range(2):
            off = base + c * chunk
            pltpu.sync_copy(tok_hbm.at[pl.ds(off, chunk)], idx_v)
            pltpu.async_copy(x_hbm.at[idx_v], rows_v, sem).wait()
            pltpu.sync_copy(rows_v, xg_hbm.at[pl.ds(off, chunk)])

    return k(x, tok_padded)


def _ffn_kernel(te_ref, xg_ref, w1_ref, w3_ref, w2_ref, tokw_ref,
                out_ref, w1b_ref, w3b_ref, w2b_ref, acc_ref):
    j = pl.program_id(0)
    tl = pl.program_id(1)
    nvalid = te_ref[NTILES]
    prev = te_ref[jnp.maximum(tl - 1, 0)]
    refresh = (tl == 0) | (te_ref[tl] != prev)

    @pl.when((j == 0) & (tl == 0))
    def _():
        out_ref[...] = jnp.zeros_like(out_ref)

    @pl.when(refresh)
    def _():
        w1b_ref[...] = w1_ref[0].astype(jnp.bfloat16)
        w3b_ref[...] = w3_ref[0].astype(jnp.bfloat16)
        w2b_ref[...] = w2_ref[0].astype(jnp.bfloat16)

    @pl.when(tl < nvalid)
    def _():
        xb = xg_ref[...].astype(jnp.bfloat16)
        g = jnp.dot(xb, w1b_ref[...], preferred_element_type=jnp.float32)
        u = jnp.dot(xb, w3b_ref[...], preferred_element_type=jnp.float32)
        g = g * jax.nn.sigmoid(g)
        h = (g * u).astype(jnp.bfloat16)
        part = jnp.dot(h, w2b_ref[...], preferred_element_type=jnp.float32)

        sl = pl.ds(tl * BT, BT)
        if NJ > 1:
            @pl.when(j == 0)
            def _():
                acc_ref[sl, :] = part.astype(jnp.bfloat16)

            @pl.when((j > 0) & (j < NJ - 1))
            def _():
                acc_ref[sl, :] += part.astype(jnp.bfloat16)

        @pl.when(j == NJ - 1)
        def _():
            if NJ == 1:
                full = part
            else:
                full = acc_ref[sl, :].astype(jnp.float32) + part
            y = full.astype(jnp.bfloat16)
            v = tokw_ref[0]
            tok = v & 0xFFFF
            wv = (v >> 16).astype(jnp.float32) * (1.0 / 16384.0)
            ti = jax.lax.broadcasted_iota(jnp.int32, (T, BT), 0)
            pt = jnp.where(ti == tok, wv, 0.0).astype(jnp.bfloat16)
            out_ref[...] += jnp.dot(pt, y, preferred_element_type=jnp.float32)


def _ffn(scalars, xg, W1, W3, W2, tokw3):
    grid_spec = pltpu.PrefetchScalarGridSpec(
        num_scalar_prefetch=1,
        grid=(NJ, NTILES),
        in_specs=[
            pl.BlockSpec((BT, DMODEL), lambda j, tl, te: (tl, 0)),
            pl.BlockSpec((1, DMODEL, BF), lambda j, tl, te: (te[tl], 0, j)),
            pl.BlockSpec((1, DMODEL, BF), lambda j, tl, te: (te[tl], 0, j)),
            pl.BlockSpec((1, BF, DMODEL), lambda j, tl, te: (te[tl], j, 0)),
            pl.BlockSpec((1, 1, BT), lambda j, tl, te: (tl, 0, 0)),
        ],
        out_specs=pl.BlockSpec((T, DMODEL), lambda j, tl, te: (0, 0)),
        scratch_shapes=[
            pltpu.VMEM((DMODEL, BF), jnp.bfloat16),
            pltpu.VMEM((DMODEL, BF), jnp.bfloat16),
            pltpu.VMEM((BF, DMODEL), jnp.bfloat16),
            pltpu.VMEM((LPAD, DMODEL), jnp.bfloat16),
        ],
    )
    return pl.pallas_call(
        _ffn_kernel,
        grid_spec=grid_spec,
        out_shape=jax.ShapeDtypeStruct((T, DMODEL), jnp.float32),
        compiler_params=pltpu.CompilerParams(
            dimension_semantics=("arbitrary", "arbitrary"),
        ),
    )(scalars, xg, W1, W3, W2, tokw3)


@jax.jit
def _moe(x, gate_W, W1, W2, W3):
    gate_w, gate_idx = _router(x, gate_W)

    eid = gate_idx.reshape(-1)
    oh = (eid[:, None] == jnp.arange(E, dtype=jnp.int32)[None, :])
    oh = oh.astype(jnp.int32)
    counts = oh.sum(axis=0)
    rank = jnp.cumsum(oh, axis=0) - oh
    rank_i = (rank * oh).sum(axis=1)
    pc = ((counts + BT - 1) // BT) * BT
    pend = jnp.cumsum(pc)
    pstart = pend - pc
    pos = (pstart[eid] + rank_i).astype(jnp.int32)
    pair_tok = jnp.arange(T * TOPK, dtype=jnp.int32) // TOPK
    wq = jnp.round(gate_w.reshape(-1) * 16384.0).astype(jnp.int32)
    packed = (wq << 16) | pair_tok
    pad_fill = jnp.arange(LPAD, dtype=jnp.int32) % T
    tokw = pad_fill.at[pos].set(packed)
    tok_padded = tokw & 0xFFFF
    nvalid = (pend[-1] // BT).astype(jnp.int32)
    te = (jnp.arange(NTILES, dtype=jnp.int32)[:, None] * BT
          >= pend[None, :]).sum(axis=1)
    te = jnp.minimum(te, E - 1).astype(jnp.int32)
    scalars = jnp.concatenate([te, nvalid[None]])
    tokw3 = tokw.reshape(NTILES, 1, BT)

    xg = _dispatch(x, tok_padded)
    return _ffn(scalars, xg, W1, W3, W2, tokw3)


def kernel(stm, gate_W, W1, W2, W3):
    b, s, h, dh = stm.shape
    x = stm.reshape(b * s, h * dh)
    out = _moe(x, gate_W, W1, W2, W3)
    return out.reshape(b, s, h, dh)


# speedup vs baseline: 1.3773x; 1.0813x over previous
import functools

import jax
import jax.numpy as jnp
from jax import lax
from jax.experimental import pallas as pl
from jax.experimental.pallas import tpu as pltpu
from jax.experimental.pallas import tpu_sc as plsc

B, S, H, D = 1, 2048, 12, 64
DMODEL = H * D
DFF = 2048
E = 8
T = B * S
TOPK = 2

BT = 512
NTILES = (T * TOPK + E * (BT - 1) + BT - 1) // BT
LPAD = NTILES * BT
BF = 1024
NJ = DFF // BF

NC, NS = 2, 16
NW = NC * NS


def _router_kernel(x_ref, gw_ref, w_ref, idx_ref):
    x = x_ref[...]
    logits = jnp.dot(x, gw_ref[...], preferred_element_type=jnp.float32)
    am1 = jnp.argmax(logits, axis=1)[:, None]
    eids = jax.lax.broadcasted_iota(jnp.int32, logits.shape, 1)
    m1 = jnp.max(logits, axis=1, keepdims=True)
    masked = jnp.where(eids == am1, -jnp.inf, logits)
    am2 = jnp.argmax(masked, axis=1)[:, None]
    m2 = jnp.max(masked, axis=1, keepdims=True)
    w1v = 1.0 / (1.0 + jnp.exp(m2 - m1))
    w_ref[...] = jnp.concatenate([w1v, 1.0 - w1v], axis=1)
    idx_ref[...] = jnp.concatenate([am1, am2], axis=1).astype(jnp.int32)


def _router(x, gate_W):
    return pl.pallas_call(
        _router_kernel,
        out_shape=(jax.ShapeDtypeStruct((T, TOPK), jnp.float32),
                   jax.ShapeDtypeStruct((T, TOPK), jnp.int32)),
    )(x, gate_W)


def _dispatch(x, pos3):
    tok_w = T // NW

    mesh = plsc.VectorSubcoreMesh(core_axis_name="c", subcore_axis_name="s")

    @functools.partial(
        pl.kernel, mesh=mesh,
        out_type=jax.ShapeDtypeStruct((LPAD, DMODEL), jnp.float32),
        scratch_types=[
            pltpu.VMEM((TOPK, tok_w), jnp.int32),
            pltpu.VMEM((tok_w, DMODEL), jnp.float32),
            pltpu.SemaphoreType.DMA,
        ],
    )
    def k(x_hbm, pos3_hbm, xg_hbm, idx_v, rows_v, sem):
        wid = lax.axis_index("s") * NC + lax.axis_index("c")
        base = wid * tok_w
        pltpu.sync_copy(pos3_hbm.at[wid], idx_v)
        pltpu.sync_copy(x_hbm.at[pl.ds(base, tok_w)], rows_v)
        c0 = pltpu.async_copy(rows_v, xg_hbm.at[idx_v.at[0]], sem)
        c1 = pltpu.async_copy(rows_v, xg_hbm.at[idx_v.at[1]], sem)
        c0.wait()
        c1.wait()

    return k(x, pos3)


def _ffn_kernel(te_ref, xg_ref, w1_ref, w3_ref, w2_ref, tokw_ref,
                out_ref, w1b_ref, w3b_ref, w2b_ref, acc_ref):
    j = pl.program_id(0)
    tl = pl.program_id(1)
    nvalid = te_ref[NTILES]
    prev = te_ref[jnp.maximum(tl - 1, 0)]
    refresh = (tl == 0) | (te_ref[tl] != prev)

    @pl.when((j == 0) & (tl == 0))
    def _():
        out_ref[...] = jnp.zeros_like(out_ref)

    @pl.when(refresh)
    def _():
        w1b_ref[...] = w1_ref[0].astype(jnp.bfloat16)
        w3b_ref[...] = w3_ref[0].astype(jnp.bfloat16)
        w2b_ref[...] = w2_ref[0].astype(jnp.bfloat16)

    @pl.when(tl < nvalid)
    def _():
        xr = xg_ref[...]
        xb = jnp.where(jnp.abs(xr) < 1e30, xr, 0.0).astype(jnp.bfloat16)
        g = jnp.dot(xb, w1b_ref[...], preferred_element_type=jnp.float32)
        u = jnp.dot(xb, w3b_ref[...], preferred_element_type=jnp.float32)
        g = g * jax.nn.sigmoid(g)
        h = (g * u).astype(jnp.bfloat16)
        part = jnp.dot(h, w2b_ref[...], preferred_element_type=jnp.float32)

        sl = pl.ds(tl * BT, BT)
        if NJ > 1:
            @pl.when(j == 0)
            def _():
                acc_ref[sl, :] = part.astype(jnp.bfloat16)

            @pl.when((j > 0) & (j < NJ - 1))
            def _():
                acc_ref[sl, :] += part.astype(jnp.bfloat16)

        @pl.when(j == NJ - 1)
        def _():
            if NJ == 1:
                full = part
            else:
                full = acc_ref[sl, :].astype(jnp.float32) + part
            y = full.astype(jnp.bfloat16)
            v = tokw_ref[0]
            tok = v & 0xFFFF
            wv = (v >> 16).astype(jnp.float32) * (1.0 / 16384.0)
            ti = jax.lax.broadcasted_iota(jnp.int32, (T, BT), 0)
            pt = jnp.where(ti == tok, wv, 0.0).astype(jnp.bfloat16)
            out_ref[...] += jnp.dot(pt, y, preferred_element_type=jnp.float32)


def _ffn(scalars, xg, W1, W3, W2, tokw3):
    grid_spec = pltpu.PrefetchScalarGridSpec(
        num_scalar_prefetch=1,
        grid=(NJ, NTILES),
        in_specs=[
            pl.BlockSpec((BT, DMODEL), lambda j, tl, te: (tl, 0)),
            pl.BlockSpec((1, DMODEL, BF), lambda j, tl, te: (te[tl], 0, j)),
            pl.BlockSpec((1, DMODEL, BF), lambda j, tl, te: (te[tl], 0, j)),
            pl.BlockSpec((1, BF, DMODEL), lambda j, tl, te: (te[tl], j, 0)),
            pl.BlockSpec((1, 1, BT), lambda j, tl, te: (tl, 0, 0)),
        ],
        out_specs=pl.BlockSpec((T, DMODEL), lambda j, tl, te: (0, 0)),
        scratch_shapes=[
            pltpu.VMEM((DMODEL, BF), jnp.bfloat16),
            pltpu.VMEM((DMODEL, BF), jnp.bfloat16),
            pltpu.VMEM((BF, DMODEL), jnp.bfloat16),
            pltpu.VMEM((LPAD, DMODEL), jnp.bfloat16),
        ],
    )
    return pl.pallas_call(
        _ffn_kernel,
        grid_spec=grid_spec,
        out_shape=jax.ShapeDtypeStruct((T, DMODEL), jnp.float32),
        compiler_params=pltpu.CompilerParams(
            dimension_semantics=("arbitrary", "arbitrary"),
        ),
    )(scalars, xg, W1, W3, W2, tokw3)


@jax.jit
def _moe(x, gate_W, W1, W2, W3):
    gate_w, gate_idx = _router(x, gate_W)

    eid = gate_idx.reshape(-1)
    oh = (eid[:, None] == jnp.arange(E, dtype=jnp.int32)[None, :])
    oh = oh.astype(jnp.int32)
    counts = oh.sum(axis=0)
    rank = jnp.cumsum(oh, axis=0) - oh
    rank_i = (rank * oh).sum(axis=1)
    pc = ((counts + BT - 1) // BT) * BT
    pend = jnp.cumsum(pc)
    pstart = pend - pc
    pos = (pstart[eid] + rank_i).astype(jnp.int32)
    pair_tok = jnp.arange(T * TOPK, dtype=jnp.int32) // TOPK
    wq = jnp.round(gate_w.reshape(-1) * 16384.0).astype(jnp.int32)
    packed = (wq << 16) | pair_tok
    pad_fill = jnp.arange(LPAD, dtype=jnp.int32) % T
    tokw = pad_fill.at[pos].set(packed)
    pos3 = pos.reshape(NW, T // NW, TOPK).transpose(0, 2, 1)
    nvalid = (pend[-1] // BT).astype(jnp.int32)
    te = (jnp.arange(NTILES, dtype=jnp.int32)[:, None] * BT
          >= pend[None, :]).sum(axis=1)
    te = jnp.minimum(te, E - 1).astype(jnp.int32)
    scalars = jnp.concatenate([te, nvalid[None]])
    tokw3 = tokw.reshape(NTILES, 1, BT)

    xg = _dispatch(x, pos3)
    return _ffn(scalars, xg, W1, W3, W2, tokw3)


def kernel(stm, gate_W, W1, W2, W3):
    b, s, h, dh = stm.shape
    x = stm.reshape(b * s, h * dh)
    out = _moe(x, gate_W, W1, W2, W3)
    return out.reshape(b, s, h, dh)


# speedup vs baseline: 1.3801x; 1.0020x over previous
import functools

import jax
import jax.numpy as jnp
from jax import lax
from jax.experimental import pallas as pl
from jax.experimental.pallas import tpu as pltpu
from jax.experimental.pallas import tpu_sc as plsc

B, S, H, D = 1, 2048, 12, 64
DMODEL = H * D
DFF = 2048
E = 8
T = B * S
TOPK = 2

BT = 512
NTILES = (T * TOPK + E * (BT - 1) + BT - 1) // BT
LPAD = NTILES * BT
BF = 1024
NJ = DFF // BF

NC, NS = 2, 16
NW = NC * NS


def _router_kernel(x_ref, gw_ref, w_ref, idx_ref):
    x = x_ref[...]
    logits = jnp.dot(x, gw_ref[...], preferred_element_type=jnp.float32)
    am1 = jnp.argmax(logits, axis=1)[:, None]
    eids = jax.lax.broadcasted_iota(jnp.int32, logits.shape, 1)
    m1 = jnp.max(logits, axis=1, keepdims=True)
    masked = jnp.where(eids == am1, -jnp.inf, logits)
    am2 = jnp.argmax(masked, axis=1)[:, None]
    m2 = jnp.max(masked, axis=1, keepdims=True)
    w1v = 1.0 / (1.0 + jnp.exp(m2 - m1))
    w_ref[...] = jnp.concatenate([w1v, 1.0 - w1v], axis=1)
    idx_ref[...] = jnp.concatenate([am1, am2], axis=1).astype(jnp.int32)


def _router(x, gate_W):
    return pl.pallas_call(
        _router_kernel,
        out_shape=(jax.ShapeDtypeStruct((T, TOPK), jnp.float32),
                   jax.ShapeDtypeStruct((T, TOPK), jnp.int32)),
    )(x, gate_W)


def _dispatch(x, pos3):
    tok_w = T // NW

    mesh = plsc.VectorSubcoreMesh(core_axis_name="c", subcore_axis_name="s")

    @functools.partial(
        pl.kernel, mesh=mesh,
        out_type=jax.ShapeDtypeStruct((LPAD, DMODEL), jnp.float32),
        scratch_types=[
            pltpu.VMEM((TOPK, tok_w), jnp.int32),
            pltpu.VMEM((tok_w, DMODEL), jnp.float32),
            pltpu.SemaphoreType.DMA,
        ],
    )
    def k(x_hbm, pos3_hbm, xg_hbm, idx_v, rows_v, sem):
        wid = lax.axis_index("s") * NC + lax.axis_index("c")
        base = wid * tok_w
        pltpu.sync_copy(pos3_hbm.at[wid], idx_v)
        pltpu.sync_copy(x_hbm.at[pl.ds(base, tok_w)], rows_v)
        c0 = pltpu.async_copy(rows_v, xg_hbm.at[idx_v.at[0]], sem)
        c1 = pltpu.async_copy(rows_v, xg_hbm.at[idx_v.at[1]], sem)
        c0.wait()
        c1.wait()

    return k(x, pos3)


def _ffn_kernel(te_ref, xg_ref, w1_ref, w3_ref, w2_ref, tokw_ref,
                out_ref, w13b_ref, w2b_ref, acc_ref):
    j = pl.program_id(0)
    tl = pl.program_id(1)
    nvalid = te_ref[NTILES]
    prev = te_ref[jnp.maximum(tl - 1, 0)]
    refresh = (tl == 0) | (te_ref[tl] != prev)

    @pl.when((j == 0) & (tl == 0))
    def _():
        out_ref[...] = jnp.zeros_like(out_ref)

    @pl.when(refresh)
    def _():
        w13b_ref[:, :BF] = w1_ref[0].astype(jnp.bfloat16)
        w13b_ref[:, BF:] = w3_ref[0].astype(jnp.bfloat16)
        w2b_ref[...] = w2_ref[0].astype(jnp.bfloat16)

    @pl.when(tl < nvalid)
    def _():
        xr = xg_ref[...]
        xb = jnp.where(jnp.abs(xr) < 1e30, xr, 0.0).astype(jnp.bfloat16)
        gu = jnp.dot(xb, w13b_ref[...], preferred_element_type=jnp.float32)
        g = gu[:, :BF]
        u = gu[:, BF:]
        g = g * jax.nn.sigmoid(g)
        h = (g * u).astype(jnp.bfloat16)
        part = jnp.dot(h, w2b_ref[...], preferred_element_type=jnp.float32)

        sl = pl.ds(tl * BT, BT)
        if NJ > 1:
            @pl.when(j == 0)
            def _():
                acc_ref[sl, :] = part.astype(jnp.bfloat16)

            @pl.when((j > 0) & (j < NJ - 1))
            def _():
                acc_ref[sl, :] += part.astype(jnp.bfloat16)

        @pl.when(j == NJ - 1)
        def _():
            if NJ == 1:
                full = part
            else:
                full = acc_ref[sl, :].astype(jnp.float32) + part
            y = full.astype(jnp.bfloat16)
            v = tokw_ref[0]
            tok = v & 0xFFFF
            wv = (v >> 16).astype(jnp.float32) * (1.0 / 16384.0)
            ti = jax.lax.broadcasted_iota(jnp.int32, (T, BT), 0)
            pt = jnp.where(ti == tok, wv, 0.0).astype(jnp.bfloat16)
            out_ref[...] += jnp.dot(pt, y, preferred_element_type=jnp.float32)


def _ffn(scalars, xg, W1, W3, W2, tokw3):
    grid_spec = pltpu.PrefetchScalarGridSpec(
        num_scalar_prefetch=1,
        grid=(NJ, NTILES),
        in_specs=[
            pl.BlockSpec((BT, DMODEL), lambda j, tl, te: (tl, 0)),
            pl.BlockSpec((1, DMODEL, BF), lambda j, tl, te: (te[tl], 0, j)),
            pl.BlockSpec((1, DMODEL, BF), lambda j, tl, te: (te[tl], 0, j)),
            pl.BlockSpec((1, BF, DMODEL), lambda j, tl, te: (te[tl], j, 0)),
            pl.BlockSpec((1, 1, BT), lambda j, tl, te: (tl, 0, 0)),
        ],
        out_specs=pl.BlockSpec((T, DMODEL), lambda j, tl, te: (0, 0)),
        scratch_shapes=[
            pltpu.VMEM((DMODEL, 2 * BF), jnp.bfloat16),
            pltpu.VMEM((BF, DMODEL), jnp.bfloat16),
            pltpu.VMEM((LPAD, DMODEL), jnp.bfloat16),
        ],
    )
    return pl.pallas_call(
        _ffn_kernel,
        grid_spec=grid_spec,
        out_shape=jax.ShapeDtypeStruct((T, DMODEL), jnp.float32),
        compiler_params=pltpu.CompilerParams(
            dimension_semantics=("arbitrary", "arbitrary"),
        ),
    )(scalars, xg, W1, W3, W2, tokw3)


@jax.jit
def _moe(x, gate_W, W1, W2, W3):
    gate_w, gate_idx = _router(x, gate_W)

    eid = gate_idx.reshape(-1)
    oh = (eid[:, None] == jnp.arange(E, dtype=jnp.int32)[None, :])
    oh = oh.astype(jnp.int32)
    counts = oh.sum(axis=0)
    rank = jnp.cumsum(oh, axis=0) - oh
    rank_i = (rank * oh).sum(axis=1)
    pc = ((counts + BT - 1) // BT) * BT
    pend = jnp.cumsum(pc)
    pstart = pend - pc
    pos = (pstart[eid] + rank_i).astype(jnp.int32)
    pair_tok = jnp.arange(T * TOPK, dtype=jnp.int32) // TOPK
    wq = jnp.round(gate_w.reshape(-1) * 16384.0).astype(jnp.int32)
    packed = (wq << 16) | pair_tok
    pad_fill = jnp.arange(LPAD, dtype=jnp.int32) % T
    tokw = pad_fill.at[pos].set(packed)
    pos3 = pos.reshape(NW, T // NW, TOPK).transpose(0, 2, 1)
    nvalid = (pend[-1] // BT).astype(jnp.int32)
    te = (jnp.arange(NTILES, dtype=jnp.int32)[:, None] * BT
          >= pend[None, :]).sum(axis=1)
    te = jnp.minimum(te, E - 1).astype(jnp.int32)
    scalars = jnp.concatenate([te, nvalid[None]])
    tokw3 = tokw.reshape(NTILES, 1, BT)

    xg = _dispatch(x, pos3)
    return _ffn(scalars, xg, W1, W3, W2, tokw3)


def kernel(stm, gate_W, W1, W2, W3):
    b, s, h, dh = stm.shape
    x = stm.reshape(b * s, h * dh)
    out = _moe(x, gate_W, W1, W2, W3)
    return out.reshape(b, s, h, dh)
